# baseline (speedup 1.0000x reference)
import functools

import jax
import jax.numpy as jnp
from jax.experimental import pallas as pl
from jax.experimental.pallas import tpu as pltpu

_MEMORY_SLOTS = 1024
_MEMORY_TOPK = 32
_RETRIEVAL_WEIGHT = 0.5


def _avg_floor(a, b):
    return (a & b) + ((a ^ b) >> 1)


def _body(xf_ref, wk_ref, bk_ref, wv_ref, bv_ref, wq_ref, bq_ref,
          wp_ref, bp_ref, wr_ref, br_ref, o_ref, k_s, v_s, *,
          tb, pool, topk, inv_sqrt_c):
    nt = (((1,), (1,)), ((), ()))

    @pl.when(pl.program_id(1) == 0)
    def _build_kv():
        xb = xf_ref[0]
        t, c = xb.shape
        s = t // pool
        if pool == 1:
            slots = xb
        else:
            rows = jax.lax.broadcasted_iota(jnp.int32, (s, t), 0)
            cols = jax.lax.broadcasted_iota(jnp.int32, (s, t), 1)
            pmat = jnp.where(cols // pool == rows, 1.0 / pool, 0.0)
            slots = jnp.dot(pmat, xb, preferred_element_type=jnp.float32)
        k_s[...] = jax.lax.dot_general(
            slots, wk_ref[...], nt,
            preferred_element_type=jnp.float32) + bk_ref[...]
        v_s[...] = jax.lax.dot_general(
            slots, wv_ref[...], nt,
            preferred_element_type=jnp.float32) + bv_ref[...]

    xt = xf_ref[0, pl.ds(pl.program_id(1) * tb, tb), :]
    q = jax.lax.dot_general(xt, wq_ref[...], nt,
                            preferred_element_type=jnp.float32) + bq_ref[...]
    scores = jax.lax.dot_general(q, k_s[...], nt,
                                 preferred_element_type=jnp.float32)
    scores = scores * inv_sqrt_c

    m = jnp.max(scores, axis=-1, keepdims=True)
    mn = jnp.min(scores, axis=-1, keepdims=True)
    d = m - scores

    lo = jnp.zeros(m.shape, dtype=jnp.int32)
    hi = jax.lax.bitcast_convert_type(m - mn, jnp.int32)
    s_slots = d.shape[-1]
    nlanes = min(128, s_slots)
    ones = jnp.ones((nlanes, 128), dtype=jnp.float32)

    def step(_, carry):
        lo, hi = carry
        mid = _avg_floor(lo, hi)
        mid_f = jax.lax.bitcast_convert_type(mid, jnp.float32)
        mask = jnp.where(d <= mid_f, 1.0, 0.0)
        part = mask[:, :nlanes]
        for j in range(nlanes, s_slots, nlanes):
            part = part + mask[:, j:j + nlanes]
        cnt = jnp.dot(part, ones,
                      preferred_element_type=jnp.float32)[:, :1]
        ge = cnt >= topk
        return jnp.where(ge, lo, mid + 1), jnp.where(ge, mid, hi)

    lo, hi = jax.lax.fori_loop(0, 31, step, (lo, hi))
    thresh = jax.lax.bitcast_convert_type(hi, jnp.float32)
    w = jnp.where(d <= thresh, jnp.exp(scores - m), 0.0)
    attn = w * (1.0 / jnp.sum(w, axis=-1, keepdims=True))

    r = jnp.dot(attn.astype(jnp.bfloat16), v_s[...].astype(jnp.bfloat16),
                preferred_element_type=jnp.float32)
    r = jax.lax.dot_general(r.astype(jnp.bfloat16),
                            wp_ref[...].astype(jnp.bfloat16), nt,
                            preferred_element_type=jnp.float32) + bp_ref[...]

    wd = wr_ref[1:2, :] - wr_ref[0:1, :]
    bd = br_ref[0:1, 1:2] - br_ref[0:1, 0:1]
    gl = jnp.sum(xt * wd, axis=-1, keepdims=True) + bd
    g = jax.nn.sigmoid(gl)

    o_ref[0] = xt + _RETRIEVAL_WEIGHT * g * r


def kernel(x, Wq, bq, Wk, bk, Wv, bv, Wp, bp, Wr, br):
    B, T, C = x.shape
    S = min(T, _MEMORY_SLOTS)
    K = min(_MEMORY_TOPK, S)
    pool = T // S

    bq2 = bq.reshape(1, C)
    bk2 = bk.reshape(1, C)
    bv2 = bv.reshape(1, C)
    bp2 = bp.reshape(1, C)
    br2 = br.reshape(1, 2)

    TB = min(512, T)
    out = pl.pallas_call(
        functools.partial(_body, tb=TB, pool=pool, topk=K,
                          inv_sqrt_c=float(1.0 / (C ** 0.5))),
        grid=(B, T // TB),
        in_specs=[
            pl.BlockSpec((1, T, C), lambda b, t: (b, 0, 0)),
            pl.BlockSpec((C, C), lambda b, t: (0, 0)),
            pl.BlockSpec((1, C), lambda b, t: (0, 0)),
            pl.BlockSpec((C, C), lambda b, t: (0, 0)),
            pl.BlockSpec((1, C), lambda b, t: (0, 0)),
            pl.BlockSpec((C, C), lambda b, t: (0, 0)),
            pl.BlockSpec((1, C), lambda b, t: (0, 0)),
            pl.BlockSpec((C, C), lambda b, t: (0, 0)),
            pl.BlockSpec((1, C), lambda b, t: (0, 0)),
            pl.BlockSpec((2, C), lambda b, t: (0, 0)),
            pl.BlockSpec((1, 2), lambda b, t: (0, 0)),
        ],
        out_specs=pl.BlockSpec((1, TB, C), lambda b, t: (b, t, 0)),
        out_shape=jax.ShapeDtypeStruct((B, T, C), jnp.float32),
        scratch_shapes=[
            pltpu.VMEM((S, C), jnp.float32),
            pltpu.VMEM((S, C), jnp.float32),
        ],
        compiler_params=pltpu.CompilerParams(
            dimension_semantics=("arbitrary", "arbitrary")),
    )(x, Wk, bk2, Wv, bv2, Wq, bq2, Wp, bp2, Wr, br2)
    return out


# speedup vs baseline: 1.3074x; 1.3074x over previous
import functools

import jax
import jax.numpy as jnp
from jax.experimental import pallas as pl
from jax.experimental.pallas import tpu as pltpu

_MEMORY_SLOTS = 1024
_MEMORY_TOPK = 32
_RETRIEVAL_WEIGHT = 0.5


def _avg_floor(a, b):
    return (a & b) + ((a ^ b) >> 1)


def _body(xf_ref, wk_ref, bk_ref, wv_ref, bv_ref, wq_ref, bq_ref,
          wp_ref, bp_ref, wr_ref, br_ref, o_ref, k_s, v_s, *,
          tb, pool, topk, inv_sqrt_c):
    nt = (((1,), (1,)), ((), ()))

    @pl.when(pl.program_id(1) == 0)
    def _build_kv():
        xb = xf_ref[0]
        t, c = xb.shape
        s = t // pool
        if pool == 1:
            slots = xb
        else:
            rows = jax.lax.broadcasted_iota(jnp.int32, (s, t), 0)
            cols = jax.lax.broadcasted_iota(jnp.int32, (s, t), 1)
            pmat = jnp.where(cols // pool == rows, 1.0 / pool, 0.0)
            slots = jnp.dot(pmat, xb, preferred_element_type=jnp.float32)
        k_s[...] = jax.lax.dot_general(
            slots, wk_ref[...], nt,
            preferred_element_type=jnp.float32) + bk_ref[...]
        v_s[...] = jax.lax.dot_general(
            slots, wv_ref[...], nt,
            preferred_element_type=jnp.float32) + bv_ref[...]

    xt = xf_ref[0, pl.ds(pl.program_id(1) * tb, tb), :]
    q = jax.lax.dot_general(xt, wq_ref[...], nt,
                            preferred_element_type=jnp.float32) + bq_ref[...]
    scores = jax.lax.dot_general(q, k_s[...], nt,
                                 preferred_element_type=jnp.float32)
    scores = scores * inv_sqrt_c

    m = jnp.max(scores, axis=-1, keepdims=True)
    mn = jnp.min(scores, axis=-1, keepdims=True)
    d = m - scores

    lo = jnp.zeros(m.shape, dtype=jnp.int32)
    hi = jax.lax.bitcast_convert_type(m - mn, jnp.int32)

    for _ in range(31):
        mid = _avg_floor(lo, hi)
        mid_f = jax.lax.bitcast_convert_type(mid, jnp.float32)
        cnt = jnp.sum(jnp.where(d <= mid_f, 1.0, 0.0),
                      axis=-1, keepdims=True)
        ge = cnt >= topk
        lo = jnp.where(ge, lo, mid + 1)
        hi = jnp.where(ge, mid, hi)
    thresh = jax.lax.bitcast_convert_type(hi, jnp.float32)
    w = jnp.where(d <= thresh, jnp.exp(scores - m), 0.0)
    attn = w * (1.0 / jnp.sum(w, axis=-1, keepdims=True))

    r = jnp.dot(attn.astype(jnp.bfloat16), v_s[...].astype(jnp.bfloat16),
                preferred_element_type=jnp.float32)
    r = jax.lax.dot_general(r.astype(jnp.bfloat16),
                            wp_ref[...].astype(jnp.bfloat16), nt,
                            preferred_element_type=jnp.float32) + bp_ref[...]

    wd = wr_ref[1:2, :] - wr_ref[0:1, :]
    bd = br_ref[0:1, 1:2] - br_ref[0:1, 0:1]
    gl = jnp.sum(xt * wd, axis=-1, keepdims=True) + bd
    g = jax.nn.sigmoid(gl)

    o_ref[0] = xt + _RETRIEVAL_WEIGHT * g * r


def kernel(x, Wq, bq, Wk, bk, Wv, bv, Wp, bp, Wr, br):
    B, T, C = x.shape
    S = min(T, _MEMORY_SLOTS)
    K = min(_MEMORY_TOPK, S)
    pool = T // S

    bq2 = bq.reshape(1, C)
    bk2 = bk.reshape(1, C)
    bv2 = bv.reshape(1, C)
    bp2 = bp.reshape(1, C)
    br2 = br.reshape(1, 2)

    TB = min(512, T)
    out = pl.pallas_call(
        functools.partial(_body, tb=TB, pool=pool, topk=K,
                          inv_sqrt_c=float(1.0 / (C ** 0.5))),
        grid=(B, T // TB),
        in_specs=[
            pl.BlockSpec((1, T, C), lambda b, t: (b, 0, 0)),
            pl.BlockSpec((C, C), lambda b, t: (0, 0)),
            pl.BlockSpec((1, C), lambda b, t: (0, 0)),
            pl.BlockSpec((C, C), lambda b, t: (0, 0)),
            pl.BlockSpec((1, C), lambda b, t: (0, 0)),
            pl.BlockSpec((C, C), lambda b, t: (0, 0)),
            pl.BlockSpec((1, C), lambda b, t: (0, 0)),
            pl.BlockSpec((C, C), lambda b, t: (0, 0)),
            pl.BlockSpec((1, C), lambda b, t: (0, 0)),
            pl.BlockSpec((2, C), lambda b, t: (0, 0)),
            pl.BlockSpec((1, 2), lambda b, t: (0, 0)),
        ],
        out_specs=pl.BlockSpec((1, TB, C), lambda b, t: (b, t, 0)),
        out_shape=jax.ShapeDtypeStruct((B, T, C), jnp.float32),
        scratch_shapes=[
            pltpu.VMEM((S, C), jnp.float32),
            pltpu.VMEM((S, C), jnp.float32),
        ],
        compiler_params=pltpu.CompilerParams(
            dimension_semantics=("arbitrary", "arbitrary")),
    )(x, Wk, bk2, Wv, bv2, Wq, bq2, Wp, bp2, Wr, br2)
    return out
